# baseline (speedup 1.0000x reference)
import functools

import jax
import jax.numpy as jnp
from jax import lax
from jax.experimental import pallas as pl
from jax.experimental.pallas import tpu as pltpu
from jax.experimental.pallas import tpu_sc as plsc

D = 64
B = 16384 * 50
NW = 32
BPW = B // NW
CHUNK = 512
NCHUNK = BPW // CHUNK
KSUB = CHUNK // 128
NPAIR = NCHUNK // 2
IDXROWS = BPW // 128

_mesh = plsc.VectorSubcoreMesh(core_axis_name="c", subcore_axis_name="s")


@functools.partial(
    pl.kernel,
    out_type=jax.ShapeDtypeStruct((B, D), jnp.float32),
    mesh=_mesh,
    scratch_types=[
        pltpu.VMEM((IDXROWS, 128), jnp.int32),
        pltpu.VMEM((CHUNK, D), jnp.float32),
        pltpu.VMEM((CHUNK, D), jnp.float32),
        pltpu.SemaphoreType.DMA,
        pltpu.SemaphoreType.DMA,
        pltpu.SemaphoreType.DMA,
        pltpu.SemaphoreType.DMA,
    ],
    compiler_params=pltpu.CompilerParams(use_tc_tiling_on_sc=False),
)
def _sc_gather(ids_hbm, table_hbm, out_hbm, idx_v, rows_a, rows_b,
               sem_ga, sem_gb, sem_wa, sem_wb):
    wid = lax.axis_index("s") * 2 + lax.axis_index("c")
    out_base = wid * BPW

    pltpu.sync_copy(ids_hbm.at[pl.ds(wid * IDXROWS, IDXROWS)], idx_v)

    def fire_gathers(ci, rows_v, sem):
        return [
            pltpu.async_copy(
                table_hbm.at[idx_v.at[ci * KSUB + j]],
                rows_v.at[pl.ds(j * 128, 128)],
                sem,
            )
            for j in range(KSUB)
        ]

    def writeout(ci, rows_v, sem):
        return pltpu.make_async_copy(
            rows_v, out_hbm.at[pl.ds(out_base + ci * CHUNK, CHUNK)], sem)

    ga = fire_gathers(0, rows_a, sem_ga)
    gb = fire_gathers(1, rows_b, sem_gb)
    for cp in ga:
        cp.wait()
    writeout(0, rows_a, sem_wa).start()
    for cp in gb:
        cp.wait()
    writeout(1, rows_b, sem_wb).start()

    def step(g, _):
        c0 = 2 * g
        c1 = c0 + 1
        writeout(c0 - 2, rows_a, sem_wa).wait()
        ga = fire_gathers(c0, rows_a, sem_ga)
        writeout(c1 - 2, rows_b, sem_wb).wait()
        gb = fire_gathers(c1, rows_b, sem_gb)
        for cp in ga:
            cp.wait()
        writeout(c0, rows_a, sem_wa).start()
        for cp in gb:
            cp.wait()
        writeout(c1, rows_b, sem_wb).start()
        return 0

    lax.fori_loop(1, NPAIR, step, 0)

    writeout(NCHUNK - 2, rows_a, sem_wa).wait()
    writeout(NCHUNK - 1, rows_b, sem_wb).wait()


def kernel(token_ids, embd_mat):
    ids = token_ids.reshape(B // 128, 128)
    out = _sc_gather(ids, embd_mat)
    return out.reshape(token_ids.shape[0], token_ids.shape[1], D)


# speedup vs baseline: 1.0104x; 1.0104x over previous
import functools

import jax
import jax.numpy as jnp
from jax import lax
from jax.experimental import pallas as pl
from jax.experimental.pallas import tpu as pltpu
from jax.experimental.pallas import tpu_sc as plsc

D = 64
B = 16384 * 50
NW = 32
BPW = B // NW
CHUNK = 256
NBUF = 4
NCHUNK = BPW // CHUNK
KSUB = CHUNK // 128
NGROUP = NCHUNK // NBUF
IDXROWS = BPW // 128

_mesh = plsc.VectorSubcoreMesh(core_axis_name="c", subcore_axis_name="s")


@functools.partial(
    pl.kernel,
    out_type=jax.ShapeDtypeStruct((B, D), jnp.float32),
    mesh=_mesh,
    scratch_types=(
        [pltpu.VMEM((IDXROWS, 128), jnp.int32)]
        + [pltpu.VMEM((CHUNK, D), jnp.float32) for _ in range(NBUF)]
        + [pltpu.SemaphoreType.DMA for _ in range(2 * NBUF)]
    ),
    compiler_params=pltpu.CompilerParams(use_tc_tiling_on_sc=False),
)
def _sc_gather(ids_hbm, table_hbm, out_hbm, idx_v, *scratch):
    bufs = scratch[:NBUF]
    gsems = scratch[NBUF:2 * NBUF]
    wsems = scratch[2 * NBUF:]

    wid = lax.axis_index("s") * 2 + lax.axis_index("c")
    out_base = wid * BPW

    pltpu.sync_copy(ids_hbm.at[pl.ds(wid * IDXROWS, IDXROWS)], idx_v)

    def fire_gathers(ci, rows_v, sem):
        return [
            pltpu.async_copy(
                table_hbm.at[idx_v.at[ci * KSUB + j]],
                rows_v.at[pl.ds(j * 128, 128)],
                sem,
            )
            for j in range(KSUB)
        ]

    def writeout(ci, rows_v, sem):
        return pltpu.make_async_copy(
            rows_v, out_hbm.at[pl.ds(out_base + ci * CHUNK, CHUNK)], sem)

    first = [fire_gathers(b, bufs[b], gsems[b]) for b in range(NBUF)]
    for b in range(NBUF):
        for cp in first[b]:
            cp.wait()
        writeout(b, bufs[b], wsems[b]).start()

    def step(g, _):
        c0 = g * NBUF
        gs = []
        for b in range(NBUF):
            writeout(c0 + b - NBUF, bufs[b], wsems[b]).wait()
            gs.append(fire_gathers(c0 + b, bufs[b], gsems[b]))
        for b in range(NBUF):
            for cp in gs[b]:
                cp.wait()
            writeout(c0 + b, bufs[b], wsems[b]).start()
        return 0

    lax.fori_loop(1, NGROUP, step, 0)

    for b in range(NBUF):
        writeout(NCHUNK - NBUF + b, bufs[b], wsems[b]).wait()


def kernel(token_ids, embd_mat):
    ids = token_ids.reshape(B // 128, 128)
    out = _sc_gather(ids, embd_mat)
    return out.reshape(token_ids.shape[0], token_ids.shape[1], D)
